# baseline (speedup 1.0000x reference)
import functools

import jax
import jax.numpy as jnp
from jax import lax
from jax.experimental import pallas as pl
from jax.experimental.pallas import tpu as pltpu
from jax.experimental.pallas import tpu_sc as plsc


_LANES = 16


@functools.cache
def _build(num_tokens: int, seq_len: int, d_model: int):
    info = plsc.get_sparse_core_info()
    nc, ns = info.num_cores, info.num_subcores
    nw = nc * ns
    per_w = num_tokens // nw
    chunk = 16
    n_chunks = per_w // chunk
    mesh = plsc.VectorSubcoreMesh(core_axis_name="c", subcore_axis_name="s")

    @functools.partial(
        pl.kernel,
        out_type=jax.ShapeDtypeStruct((num_tokens, d_model), jnp.float32),
        mesh=mesh,
        scratch_types=[
            pltpu.VMEM((per_w,), jnp.int32),
            pltpu.VMEM((chunk, d_model), jnp.float32),
            pltpu.VMEM((chunk, d_model), jnp.float32),
            pltpu.VMEM((chunk, d_model), jnp.float32),
            pltpu.VMEM((chunk, d_model), jnp.float32),
            pltpu.SemaphoreType.DMA,
            pltpu.SemaphoreType.DMA,
            pltpu.SemaphoreType.DMA,
            pltpu.SemaphoreType.DMA,
            pltpu.SemaphoreType.DMA,
            pltpu.SemaphoreType.DMA,
        ],
    )
    def emb(x_hbm, tok_hbm, pos_hbm, out_hbm, idx_v,
            rows0, rows1, pos0, pos1, gs0, gs1, ps0, ps1, ss0, ss1):
        rows = (rows0, rows1)
        pos = (pos0, pos1)
        gsem = (gs0, gs1)
        psem = (ps0, ps1)
        ssem = (ss0, ss1)
        wid = lax.axis_index("s") * nc + lax.axis_index("c")
        base = wid * per_w
        pos_base = lax.rem(base, seq_len)
        pltpu.sync_copy(x_hbm.at[pl.ds(base, per_w)], idx_v)

        def start(ch):
            b = ch % 2
            g = pltpu.async_copy(
                tok_hbm.at[idx_v.at[pl.ds(ch * chunk, chunk)]], rows[b], gsem[b]
            )
            p = pltpu.async_copy(
                pos_hbm.at[pl.ds(pos_base + ch * chunk, chunk)], pos[b], psem[b]
            )
            return g, p

        inflight = {0: start(0)}
        stores = {}
        for ch in range(n_chunks):
            b = ch % 2
            if ch + 1 < n_chunks:
                if ch - 1 in stores:
                    stores.pop(ch - 1).wait()
                inflight[ch + 1] = start(ch + 1)
            g, p = inflight.pop(ch)
            g.wait()
            p.wait()

            @plsc.parallel_loop(0, chunk * d_model, _LANES, unroll=8)
            def _(i):
                r = i // d_model
                col = i % d_model
                plsc.addupdate(
                    pos[b].at[r, pl.ds(col, _LANES)], rows[b][r, pl.ds(col, _LANES)]
                )

            stores[ch] = pltpu.async_copy(
                pos[b], out_hbm.at[pl.ds(base + ch * chunk, chunk)], ssem[b]
            )
        for ch in sorted(stores):
            stores.pop(ch).wait()

    return emb


def kernel(x, token_table, pos_table):
    batch, seq_len = x.shape
    d_model = token_table.shape[1]
    emb = _build(batch * seq_len, seq_len, d_model)
    out = emb(x.reshape(-1).astype(jnp.int32), token_table, pos_table)
    return out.reshape(batch, seq_len, d_model)


# speedup vs baseline: 1.0622x; 1.0622x over previous
import functools

import jax
import jax.numpy as jnp
from jax import lax
from jax.experimental import pallas as pl
from jax.experimental.pallas import tpu as pltpu
from jax.experimental.pallas import tpu_sc as plsc


_LANES = 16


@functools.cache
def _build(num_tokens: int, seq_len: int, d_model: int):
    info = plsc.get_sparse_core_info()
    nc, ns = info.num_cores, info.num_subcores
    nw = nc * ns
    per_w = num_tokens // nw
    chunk = 16
    n_chunks = per_w // chunk
    n_pairs = n_chunks // 2
    mesh = plsc.VectorSubcoreMesh(core_axis_name="c", subcore_axis_name="s")

    @functools.partial(
        pl.kernel,
        out_type=jax.ShapeDtypeStruct((num_tokens, d_model), jnp.float32),
        mesh=mesh,
        scratch_types=[
            pltpu.VMEM((per_w,), jnp.int32),
            pltpu.VMEM((chunk, d_model), jnp.float32),
            pltpu.VMEM((chunk, d_model), jnp.float32),
            pltpu.VMEM((chunk, d_model), jnp.float32),
            pltpu.VMEM((chunk, d_model), jnp.float32),
            pltpu.VMEM((chunk, d_model), jnp.float32),
            pltpu.VMEM((chunk, d_model), jnp.float32),
            pltpu.SemaphoreType.DMA,
            pltpu.SemaphoreType.DMA,
            pltpu.SemaphoreType.DMA,
            pltpu.SemaphoreType.DMA,
            pltpu.SemaphoreType.DMA,
            pltpu.SemaphoreType.DMA,
        ],
    )
    def emb(x_hbm, tok_hbm, pos_hbm, out_hbm, idx_v,
            rows0, rows1, pos0, pos1, st0, st1,
            gs0, gs1, ps0, ps1, ss0, ss1):
        rows = (rows0, rows1)
        pos = (pos0, pos1)
        stage = (st0, st1)
        gsem = (gs0, gs1)
        psem = (ps0, ps1)
        ssem = (ss0, ss1)
        wid = lax.axis_index("s") * nc + lax.axis_index("c")
        base = wid * per_w
        pos_base = lax.rem(base, seq_len)
        pltpu.sync_copy(x_hbm.at[pl.ds(base, per_w)], idx_v)

        def gather_desc(c, p):
            return pltpu.make_async_copy(
                tok_hbm.at[idx_v.at[pl.ds(c * chunk, chunk)]], rows[p], gsem[p]
            )

        def pos_desc(c, p):
            return pltpu.make_async_copy(
                pos_hbm.at[pl.ds(pos_base + c * chunk, chunk)], pos[p], psem[p]
            )

        def store_desc(c, p):
            return pltpu.make_async_copy(
                stage[p], out_hbm.at[pl.ds(base + c * chunk, chunk)], ssem[p]
            )

        gather_desc(0, 0).start()
        pos_desc(0, 0).start()
        gather_desc(1, 1).start()
        pos_desc(1, 1).start()

        @pl.loop(0, n_pairs)
        def _(j):
            for p in range(2):
                c = 2 * j + p
                gather_desc(c, p).wait()
                pos_desc(c, p).wait()

                @pl.when(j > 0)
                def _():
                    store_desc(c - 2, p).wait()

                @plsc.parallel_loop(0, chunk * d_model, _LANES, unroll=8)
                def _(i):
                    r = i // d_model
                    col = i % d_model
                    stage[p][r, pl.ds(col, _LANES)] = (
                        rows[p][r, pl.ds(col, _LANES)]
                        + pos[p][r, pl.ds(col, _LANES)]
                    )

                store_desc(c, p).start()

                @pl.when(c + 2 < n_chunks)
                def _():
                    gather_desc(c + 2, p).start()
                    pos_desc(c + 2, p).start()

        for p in range(2):
            store_desc(n_chunks - 2 + p, p).wait()

    return emb


def kernel(x, token_table, pos_table):
    batch, seq_len = x.shape
    d_model = token_table.shape[1]
    emb = _build(batch * seq_len, seq_len, d_model)
    out = emb(x.reshape(-1).astype(jnp.int32), token_table, pos_table)
    return out.reshape(batch, seq_len, d_model)


# speedup vs baseline: 1.2345x; 1.1623x over previous
import functools

import jax
import jax.numpy as jnp
from jax import lax
from jax.experimental import pallas as pl
from jax.experimental.pallas import tpu as pltpu
from jax.experimental.pallas import tpu_sc as plsc


_LANES = 16


@functools.cache
def _build(batch: int, seq_len: int, d_model: int):
    info = plsc.get_sparse_core_info()
    nc, ns = info.num_cores, info.num_subcores
    nw = nc * ns
    pos_per_w = seq_len // nw
    chunk = 16
    n_qq = pos_per_w // chunk
    n_chunks = n_qq * batch
    mesh = plsc.VectorSubcoreMesh(core_axis_name="c", subcore_axis_name="s")

    @functools.partial(
        pl.kernel,
        out_type=jax.ShapeDtypeStruct((batch, seq_len, d_model), jnp.float32),
        mesh=mesh,
        scratch_types=[
            pltpu.VMEM((batch, pos_per_w), jnp.int32),
            pltpu.VMEM((2 * chunk, d_model), jnp.float32),
            pltpu.VMEM((2 * chunk, d_model), jnp.float32),
            pltpu.VMEM((2 * chunk, d_model), jnp.float32),
            pltpu.SemaphoreType.DMA,
            pltpu.SemaphoreType.DMA,
            pltpu.SemaphoreType.DMA,
        ],
    )
    def emb(x_hbm, tok_hbm, pos_hbm, out_hbm, idx_v, rows_v, pos_v, stage_v,
            gsem, psem, ssem):
        wid = lax.axis_index("s") * nc + lax.axis_index("c")
        s0 = wid * pos_per_w
        for bb in range(batch):
            pltpu.sync_copy(x_hbm.at[bb, pl.ds(s0, pos_per_w)], idx_v.at[bb])

        def gather_desc(t):
            qq, bb = t // batch, t % batch
            half = (t % 2) * chunk
            return pltpu.make_async_copy(
                tok_hbm.at[idx_v.at[bb, pl.ds(qq * chunk, chunk)]],
                rows_v.at[pl.ds(half, chunk)],
                gsem,
            )

        def pos_desc(qq):
            half = (qq % 2) * chunk
            return pltpu.make_async_copy(
                pos_hbm.at[pl.ds(s0 + qq * chunk, chunk)],
                pos_v.at[pl.ds(half, chunk)],
                psem,
            )

        def store_desc(t):
            qq, bb = t // batch, t % batch
            half = (t % 2) * chunk
            return pltpu.make_async_copy(
                stage_v.at[pl.ds(half, chunk)],
                out_hbm.at[bb, pl.ds(s0 + qq * chunk, chunk)],
                ssem,
            )

        gather_desc(0).start()
        gather_desc(1).start()
        pos_desc(0).start()

        @pl.loop(0, n_chunks)
        def _(t):
            qq = t // batch
            bb = lax.rem(t, batch)
            half = lax.rem(t, 2) * chunk
            phalf = lax.rem(qq, 2) * chunk

            @pl.when(bb == 0)
            def _():
                pos_desc(qq).wait()

            gather_desc(t).wait()

            @pl.when(t >= 2)
            def _():
                store_desc(t - 2).wait()

            @plsc.parallel_loop(0, chunk * d_model, _LANES, unroll=8)
            def _(i):
                r = i // d_model
                col = i % d_model
                stage_v[half + r, pl.ds(col, _LANES)] = (
                    rows_v[half + r, pl.ds(col, _LANES)]
                    + pos_v[phalf + r, pl.ds(col, _LANES)]
                )

            store_desc(t).start()

            @pl.when(t + 2 < n_chunks)
            def _():
                gather_desc(t + 2).start()

            @pl.when(jnp.logical_and(bb == 1, qq + 1 < n_qq))
            def _():
                pos_desc(qq + 1).start()

        store_desc(n_chunks - 2).wait()
        store_desc(n_chunks - 1).wait()

    return emb


def kernel(x, token_table, pos_table):
    batch, seq_len = x.shape
    d_model = token_table.shape[1]
    emb = _build(batch, seq_len, d_model)
    return emb(x.astype(jnp.int32), token_table, pos_table)


# speedup vs baseline: 1.2613x; 1.0217x over previous
import functools

import jax
import jax.numpy as jnp
from jax import lax
from jax.experimental import pallas as pl
from jax.experimental.pallas import tpu as pltpu
from jax.experimental.pallas import tpu_sc as plsc


_LANES = 16


@functools.cache
def _build(batch: int, seq_len: int, d_model: int):
    info = plsc.get_sparse_core_info()
    nc, ns = info.num_cores, info.num_subcores
    nw = nc * ns
    pos_per_w = seq_len // nw
    chunk = 16
    n_qq = pos_per_w // chunk
    n_chunks = n_qq * batch
    mesh = plsc.VectorSubcoreMesh(core_axis_name="c", subcore_axis_name="s")

    @functools.partial(
        pl.kernel,
        out_type=jax.ShapeDtypeStruct((batch, seq_len, d_model), jnp.float32),
        mesh=mesh,
        scratch_types=[
            pltpu.VMEM((batch, pos_per_w), jnp.int32),
            pltpu.VMEM((3 * chunk, d_model), jnp.float32),
            pltpu.VMEM((2 * chunk, d_model), jnp.float32),
            pltpu.VMEM((2 * chunk, d_model), jnp.float32),
            pltpu.SemaphoreType.DMA,
            pltpu.SemaphoreType.DMA,
            pltpu.SemaphoreType.DMA,
        ],
    )
    def emb(x_hbm, tok_hbm, pos_hbm, out_hbm, idx_v, rows_v, pos_v, stage_v,
            gsem, psem, ssem):
        wid = lax.axis_index("s") * nc + lax.axis_index("c")
        s0 = wid * pos_per_w
        for bb in range(batch):
            pltpu.sync_copy(x_hbm.at[bb, pl.ds(s0, pos_per_w)], idx_v.at[bb])

        def gather_desc(t):
            qq, bb = t // batch, t % batch
            slot = (t % 3) * chunk
            return pltpu.make_async_copy(
                tok_hbm.at[idx_v.at[bb, pl.ds(qq * chunk, chunk)]],
                rows_v.at[pl.ds(slot, chunk)],
                gsem,
            )

        def pos_desc(qq):
            half = (qq % 2) * chunk
            return pltpu.make_async_copy(
                pos_hbm.at[pl.ds(s0 + qq * chunk, chunk)],
                pos_v.at[pl.ds(half, chunk)],
                psem,
            )

        def store_desc(t):
            qq, bb = t // batch, t % batch
            half = (t % 2) * chunk
            return pltpu.make_async_copy(
                stage_v.at[pl.ds(half, chunk)],
                out_hbm.at[bb, pl.ds(s0 + qq * chunk, chunk)],
                ssem,
            )

        gather_desc(0).start()
        gather_desc(1).start()
        gather_desc(2).start()
        pos_desc(0).start()

        @pl.loop(0, n_chunks)
        def _(t):
            qq = t // batch
            bb = lax.rem(t, batch)
            gslot = lax.rem(t, 3) * chunk
            half = lax.rem(t, 2) * chunk
            phalf = lax.rem(qq, 2) * chunk

            @pl.when(bb == 0)
            def _():
                pos_desc(qq).wait()

            gather_desc(t).wait()

            @pl.when(t >= 2)
            def _():
                store_desc(t - 2).wait()

            @plsc.parallel_loop(0, chunk * d_model, _LANES, unroll=8)
            def _(i):
                r = i // d_model
                col = i % d_model
                stage_v[half + r, pl.ds(col, _LANES)] = (
                    rows_v[gslot + r, pl.ds(col, _LANES)]
                    + pos_v[phalf + r, pl.ds(col, _LANES)]
                )

            store_desc(t).start()

            @pl.when(t + 3 < n_chunks)
            def _():
                gather_desc(t + 3).start()

            @pl.when(jnp.logical_and(bb == 1, qq + 1 < n_qq))
            def _():
                pos_desc(qq + 1).start()

        store_desc(n_chunks - 2).wait()
        store_desc(n_chunks - 1).wait()

    return emb


def kernel(x, token_table, pos_table):
    batch, seq_len = x.shape
    d_model = token_table.shape[1]
    emb = _build(batch, seq_len, d_model)
    return emb(x.astype(jnp.int32), token_table, pos_table)


# speedup vs baseline: 1.2845x; 1.0184x over previous
import functools

import jax
import jax.numpy as jnp
from jax import lax
from jax.experimental import pallas as pl
from jax.experimental.pallas import tpu as pltpu
from jax.experimental.pallas import tpu_sc as plsc


_LANES = 16


@functools.cache
def _build(batch: int, seq_len: int, d_model: int):
    info = plsc.get_sparse_core_info()
    nc, ns = info.num_cores, info.num_subcores
    nw = nc * ns
    pos_per_w = seq_len // nw
    chunk = 8
    rows_ring = 6
    stage_ring = 4
    n_qq = pos_per_w // chunk
    n_chunks = n_qq * batch
    mesh = plsc.VectorSubcoreMesh(core_axis_name="c", subcore_axis_name="s")

    @functools.partial(
        pl.kernel,
        out_type=jax.ShapeDtypeStruct((batch, seq_len, d_model), jnp.float32),
        mesh=mesh,
        scratch_types=[
            pltpu.VMEM((batch, pos_per_w), jnp.int32),
            pltpu.VMEM((rows_ring * chunk, d_model), jnp.float32),
            pltpu.VMEM((2 * chunk, d_model), jnp.float32),
            pltpu.VMEM((stage_ring * chunk, d_model), jnp.float32),
            pltpu.SemaphoreType.DMA,
            pltpu.SemaphoreType.DMA,
            pltpu.SemaphoreType.DMA,
        ],
    )
    def emb(x_hbm, tok_hbm, pos_hbm, out_hbm, idx_v, rows_v, pos_v, stage_v,
            gsem, psem, ssem):
        wid = lax.axis_index("s") * nc + lax.axis_index("c")
        s0 = wid * pos_per_w
        for bb in range(batch):
            pltpu.sync_copy(x_hbm.at[bb, pl.ds(s0, pos_per_w)], idx_v.at[bb])

        def gather_desc(t):
            qq, bb = t // batch, t % batch
            slot = (t % rows_ring) * chunk
            return pltpu.make_async_copy(
                tok_hbm.at[idx_v.at[bb, pl.ds(qq * chunk, chunk)]],
                rows_v.at[pl.ds(slot, chunk)],
                gsem,
            )

        def pos_desc(qq):
            half = (qq % 2) * chunk
            return pltpu.make_async_copy(
                pos_hbm.at[pl.ds(s0 + qq * chunk, chunk)],
                pos_v.at[pl.ds(half, chunk)],
                psem,
            )

        def store_desc(t):
            qq, bb = t // batch, t % batch
            half = (t % stage_ring) * chunk
            return pltpu.make_async_copy(
                stage_v.at[pl.ds(half, chunk)],
                out_hbm.at[bb, pl.ds(s0 + qq * chunk, chunk)],
                ssem,
            )

        for t0 in range(rows_ring):
            gather_desc(t0).start()
        pos_desc(0).start()

        @pl.loop(0, n_chunks)
        def _(t):
            qq = t // batch
            bb = lax.rem(t, batch)
            gslot = lax.rem(t, rows_ring) * chunk
            half = lax.rem(t, stage_ring) * chunk
            phalf = lax.rem(qq, 2) * chunk

            @pl.when(bb == 0)
            def _():
                pos_desc(qq).wait()

            gather_desc(t).wait()

            @pl.when(t >= stage_ring)
            def _():
                store_desc(t - stage_ring).wait()

            @plsc.parallel_loop(0, chunk * d_model, _LANES, unroll=8)
            def _(i):
                r = i // d_model
                col = i % d_model
                stage_v[half + r, pl.ds(col, _LANES)] = (
                    rows_v[gslot + r, pl.ds(col, _LANES)]
                    + pos_v[phalf + r, pl.ds(col, _LANES)]
                )

            store_desc(t).start()

            @pl.when(t + rows_ring < n_chunks)
            def _():
                gather_desc(t + rows_ring).start()

            @pl.when(jnp.logical_and(bb == 1, qq + 1 < n_qq))
            def _():
                pos_desc(qq + 1).start()

        for t0 in range(n_chunks - stage_ring, n_chunks):
            store_desc(t0).wait()

    return emb


def kernel(x, token_table, pos_table):
    batch, seq_len = x.shape
    d_model = token_table.shape[1]
    emb = _build(batch, seq_len, d_model)
    return emb(x.astype(jnp.int32), token_table, pos_table)
